# baseline (speedup 1.0000x reference)
import functools

import jax
import jax.numpy as jnp
from jax import lax
from jax.experimental import pallas as pl
from jax.experimental.pallas import tpu as pltpu
from jax.experimental.pallas import tpu_sc as plsc

K = 8192
D = 64
N_TOK = 8192
TOK_TILE = 512
K_TILE = 1024
N_TOK_TILES = N_TOK // TOK_TILE
N_K_TILES = K // K_TILE


def _prep_body(emb_ref, w_ref, b_ref, qcm2_ref, qcpad_ref):
    res = lax.dot_general(
        emb_ref[...], w_ref[...], (((1,), (1,)), ((), ())),
        preferred_element_type=jnp.float32,
    ) + b_ref[...]
    qcm2_ref[...] = -2.0 * res
    qcpad_ref[...] = jnp.concatenate(
        [res, jnp.zeros((K, D), jnp.float32)], axis=1)


def _dist_body(zf_ref, qc_ref, idx_ref, loss_ref,
               rmin_ref, rarg_ref, acc_ref):
    t = pl.program_id(0)
    k = pl.program_id(1)

    @pl.when(k == 0)
    def _start_row():
        rmin_ref[...] = jnp.full((TOK_TILE,), jnp.inf, jnp.float32)
        rarg_ref[...] = jnp.zeros((TOK_TILE,), jnp.int32)

    z = zf_ref[...]
    qtm2 = qc_ref[...]
    cnorm = 0.25 * jnp.sum(qtm2 * qtm2, axis=1, keepdims=True)
    dots_m2 = lax.dot_general(
        qtm2, z, (((1,), (1,)), ((), ())),
        preferred_element_type=jnp.float32,
    )
    scores = cnorm + dots_m2
    lmin = jnp.min(scores, axis=0)
    larg = jnp.argmin(scores, axis=0).astype(jnp.int32) + k * K_TILE
    better = lmin < rmin_ref[...]
    rarg_ref[...] = jnp.where(better, larg, rarg_ref[...])
    rmin_ref[...] = jnp.where(better, lmin, rmin_ref[...])

    @pl.when(k == N_K_TILES - 1)
    def _end_row():
        idx_ref[0, 0, :] = rarg_ref[...]
        z2 = zf_ref[...]

        @pl.when(t == 0)
        def _init():
            acc_ref[0, 0] = 0.0

        acc_ref[0, 0] += jnp.sum(rmin_ref[...]) + jnp.sum(z2 * z2)

        @pl.when(t == N_TOK_TILES - 1)
        def _fin():
            loss_ref[0, 0] = acc_ref[0, 0] * (1.25 / float(N_TOK * D))


@functools.cache
def _make_sc_gather():
    info = plsc.get_sparse_core_info()
    nc, ns = info.num_cores, info.num_subcores
    nw = nc * ns
    b_per_w = N_TOK // nw
    n_chunks = b_per_w // 128
    mesh = plsc.VectorSubcoreMesh(core_axis_name="c", subcore_axis_name="s")

    @functools.partial(
        pl.kernel, mesh=mesh,
        out_type=jax.ShapeDtypeStruct((N_TOK, 2 * D), jnp.float32),
        scratch_types=[
            pltpu.VMEM((n_chunks, 128), jnp.int32),
            pltpu.VMEM((b_per_w, 2 * D), jnp.float32),
            pltpu.SemaphoreType.DMA,
        ],
    )
    def gather(table_hbm, idx_hbm, out_hbm, idx_v, rows_v, sem):
        wid = lax.axis_index("s") * nc + lax.axis_index("c")
        pltpu.sync_copy(idx_hbm.at[pl.ds(wid * n_chunks, n_chunks)], idx_v)
        copies = [
            pltpu.async_copy(
                table_hbm.at[idx_v.at[j]],
                rows_v.at[pl.ds(j * 128, 128)], sem)
            for j in range(n_chunks)
        ]
        for c in copies:
            c.wait()
        pltpu.sync_copy(rows_v, out_hbm.at[pl.ds(wid * b_per_w, b_per_w)])

    return gather


def kernel(z, emb_weight, proj_w, proj_b):
    zf = z.reshape(N_TOK, D)

    qc_m2, qc_pad = pl.pallas_call(
        _prep_body,
        out_shape=[
            jax.ShapeDtypeStruct((K, D), jnp.float32),
            jax.ShapeDtypeStruct((K, 2 * D), jnp.float32),
        ],
    )(emb_weight, proj_w, proj_b.reshape(1, D))

    idx3, loss = pl.pallas_call(
        _dist_body,
        grid=(N_TOK_TILES, N_K_TILES),
        in_specs=[
            pl.BlockSpec((TOK_TILE, D), lambda t, k: (t, 0)),
            pl.BlockSpec((K_TILE, D), lambda t, k: (k, 0)),
        ],
        out_specs=[
            pl.BlockSpec((1, 1, TOK_TILE), lambda t, k: (t, 0, 0)),
            pl.BlockSpec(memory_space=pltpu.SMEM),
        ],
        out_shape=[
            jax.ShapeDtypeStruct((N_TOK_TILES, 1, TOK_TILE), jnp.int32),
            jax.ShapeDtypeStruct((1, 1), jnp.float32),
        ],
        scratch_shapes=[
            pltpu.VMEM((TOK_TILE,), jnp.float32),
            pltpu.VMEM((TOK_TILE,), jnp.int32),
            pltpu.SMEM((1, 1), jnp.float32),
        ],
    )(zf, qc_m2)

    quant = _make_sc_gather()(qc_pad, idx3.reshape(N_TOK // 128, 128))
    return (
        quant[:, :D].reshape(z.shape),
        loss.reshape(()),
        idx3.reshape(z.shape[0], z.shape[1]),
    )
